# baseline (device time: 15106 ns/iter reference)
import jax
import jax.numpy as jnp
from jax import lax
from jax.experimental import pallas as pl
from jax.experimental.pallas import tpu as pltpu

N_DEV = 4
BN = 256


def kernel(x):
    m, n = x.shape

    def body(x_blk, x_any, out_blk, halo_top, halo_bot, send_sems, recv_sems):
        j = pl.program_id(0)
        my = lax.axis_index("i")
        left = (my - 1) % N_DEV
        right = (my + 1) % N_DEV

        send_left = pltpu.make_async_remote_copy(
            src_ref=x_any.at[pl.ds(0, 1)],
            dst_ref=halo_bot,
            send_sem=send_sems.at[0],
            recv_sem=recv_sems.at[0],
            device_id=(left,),
            device_id_type=pl.DeviceIdType.MESH,
        )
        send_right = pltpu.make_async_remote_copy(
            src_ref=x_any.at[pl.ds(m - 1, 1)],
            dst_ref=halo_top,
            send_sem=send_sems.at[1],
            recv_sem=recv_sems.at[1],
            device_id=(right,),
            device_id_type=pl.DeviceIdType.MESH,
        )

        @pl.when(j == 0)
        def _():
            barrier_sem = pltpu.get_barrier_semaphore()
            for nbr in (left, right):
                pl.semaphore_signal(
                    barrier_sem, inc=1,
                    device_id=(nbr,), device_id_type=pl.DeviceIdType.MESH,
                )
            pl.semaphore_wait(barrier_sem, 2)
            send_left.start()
            send_right.start()

        out_blk[pl.ds(1, m - 2), :] = (
            0.25 * x_blk[pl.ds(0, m - 2), :]
            + 0.5 * x_blk[pl.ds(1, m - 2), :]
            + 0.25 * x_blk[pl.ds(2, m - 2), :]
        )

        @pl.when(j == 0)
        def _():
            send_left.wait()
            send_right.wait()

        top_row = x_blk[pl.ds(0, 1), :]
        out_blk[pl.ds(0, 1), :] = jnp.where(
            my == 0,
            top_row,
            0.25 * halo_top[pl.ds(0, 1), pl.ds(j * BN, BN)]
            + 0.5 * top_row
            + 0.25 * x_blk[pl.ds(1, 1), :],
        )
        bot_row = x_blk[pl.ds(m - 1, 1), :]
        out_blk[pl.ds(m - 1, 1), :] = jnp.where(
            my == N_DEV - 1,
            bot_row,
            0.25 * x_blk[pl.ds(m - 2, 1), :]
            + 0.5 * bot_row
            + 0.25 * halo_bot[pl.ds(0, 1), pl.ds(j * BN, BN)],
        )

    return pl.pallas_call(
        body,
        grid=(n // BN,),
        out_shape=jax.ShapeDtypeStruct((m, n), x.dtype),
        in_specs=[
            pl.BlockSpec((m, BN), lambda j: (0, j)),
            pl.BlockSpec(memory_space=pl.ANY),
        ],
        out_specs=pl.BlockSpec((m, BN), lambda j: (0, j)),
        scratch_shapes=[
            pltpu.VMEM((1, n), x.dtype),
            pltpu.VMEM((1, n), x.dtype),
            pltpu.SemaphoreType.DMA((2,)),
            pltpu.SemaphoreType.DMA((2,)),
        ],
        compiler_params=pltpu.CompilerParams(collective_id=0),
    )(x, x)
